# baseline (speedup 1.0000x reference)
import jax
import jax.numpy as jnp
from jax.experimental import pallas as pl
from jax.experimental.pallas import tpu as pltpu

_BR = 2048
_BC = 512


def _add_body(x_ref, emb_ref, out_ref):
    out_ref[0] = x_ref[0] + emb_ref[...]


def kernel(x, emb_weight):
    batch, num_patches, dim = x.shape
    nb = num_patches // _BR
    nc = dim // _BC
    return pl.pallas_call(
        _add_body,
        grid=(nc, nb, batch),
        in_specs=[
            pl.BlockSpec((1, _BR, _BC), lambda c, i, b: (b, i, c)),
            pl.BlockSpec((_BR, _BC), lambda c, i, b: (i, c)),
        ],
        out_specs=pl.BlockSpec((1, _BR, _BC), lambda c, i, b: (b, i, c)),
        out_shape=jax.ShapeDtypeStruct(x.shape, x.dtype),
        compiler_params=pltpu.CompilerParams(
            dimension_semantics=("arbitrary", "arbitrary", "arbitrary"),
        ),
    )(x, emb_weight)
